# baseline (device time: 75746 ns/iter reference)
import jax
import jax.numpy as jnp
from jax import lax
from jax.experimental import pallas as pl
from jax.experimental.pallas import tpu as pltpu

N_DEV = 16
MBLK = 256
NBLK = 512


def kernel(x, w_mat, scale_x, scale_w):
    m, k_shard = x.shape
    k_full, n = w_mat.shape
    n_steps = n // NBLK
    assert m == N_DEV * MBLK and k_shard == MBLK and n % NBLK == 0

    xq = x.astype(jnp.float8_e5m2)

    def body(xq_ref, w_ref, sx_ref, sw_ref, out_ref,
             xg_ref, xb_ref, send_sems, recv_sems):
        j = pl.program_id(0)
        my_i = lax.axis_index("i")

        @pl.when(j == 0)
        def _comm():
            for d in range(1, N_DEV):
                dst = lax.rem(my_i + d, N_DEV)
                rdma = pltpu.make_async_remote_copy(
                    src_ref=xq_ref.at[pl.ds(dst * MBLK, MBLK), :],
                    dst_ref=xg_ref.at[:, pl.ds(my_i * MBLK, MBLK)],
                    send_sem=send_sems.at[d],
                    recv_sem=recv_sems.at[d],
                    device_id=(dst,),
                    device_id_type=pl.DeviceIdType.MESH,
                )
                rdma.start()
            xg_ref[:, pl.ds(my_i * MBLK, MBLK)] = xq_ref[pl.ds(my_i * MBLK, MBLK), :]
            for d in range(1, N_DEV):
                src = lax.rem(my_i + (N_DEV - d), N_DEV)
                recv = pltpu.make_async_remote_copy(
                    src_ref=xq_ref.at[pl.ds(src * MBLK, MBLK), :],
                    dst_ref=xg_ref.at[:, pl.ds(src * MBLK, MBLK)],
                    send_sem=send_sems.at[d],
                    recv_sem=recv_sems.at[d],
                    device_id=(src,),
                    device_id_type=pl.DeviceIdType.MESH,
                )
                recv.wait_recv()
            xb_ref[...] = xg_ref[...].astype(jnp.bfloat16)

        wb = w_ref[...].astype(jnp.bfloat16)
        acc = jnp.dot(xb_ref[...], wb, preferred_element_type=jnp.float32)
        s = sx_ref[0] * sw_ref[0]
        y = acc * s
        out_ref[...] = y * jax.nn.sigmoid(jnp.clip(y, -60.0, 60.0))

        @pl.when(j == n_steps - 1)
        def _drain():
            for d in range(1, N_DEV):
                dst = lax.rem(my_i + d, N_DEV)
                send = pltpu.make_async_remote_copy(
                    src_ref=xq_ref.at[pl.ds(dst * MBLK, MBLK), :],
                    dst_ref=xg_ref.at[:, pl.ds(my_i * MBLK, MBLK)],
                    send_sem=send_sems.at[d],
                    recv_sem=recv_sems.at[d],
                    device_id=(dst,),
                    device_id_type=pl.DeviceIdType.MESH,
                )
                send.wait_send()

    return pl.pallas_call(
        body,
        grid=(n_steps,),
        out_shape=jax.ShapeDtypeStruct((MBLK, n), jnp.float32),
        in_specs=[
            pl.BlockSpec((m, k_shard), lambda j: (0, 0)),
            pl.BlockSpec((k_full, NBLK), lambda j: (0, j)),
            pl.BlockSpec(memory_space=pltpu.SMEM),
            pl.BlockSpec(memory_space=pltpu.SMEM),
        ],
        out_specs=pl.BlockSpec((MBLK, NBLK), lambda j: (0, j)),
        scratch_shapes=[
            pltpu.VMEM((MBLK, k_full), jnp.float8_e5m2),
            pltpu.VMEM((MBLK, k_full), jnp.bfloat16),
            pltpu.SemaphoreType.DMA((N_DEV,)),
            pltpu.SemaphoreType.DMA((N_DEV,)),
        ],
        compiler_params=pltpu.CompilerParams(
            dimension_semantics=("arbitrary",),
            vmem_limit_bytes=60 * 1024 * 1024,
        ),
    )(xq, w_mat, scale_x, scale_w)


# device time: 69542 ns/iter; 1.0892x vs baseline; 1.0892x over previous
import jax
import jax.numpy as jnp
from jax import lax
from jax.experimental import pallas as pl
from jax.experimental.pallas import tpu as pltpu

N_DEV = 16
MBLK = 256
NBLK = 512


def kernel(x, w_mat, scale_x, scale_w):
    m, k_shard = x.shape
    k_full, n = w_mat.shape
    n_steps = n // NBLK
    assert m == N_DEV * MBLK and k_shard == MBLK and n % NBLK == 0

    def body(x_ref, w_ref, sx_ref, sw_ref, out_ref,
             xq_ref, xg_ref, xb_ref, send_sems, recv_sems):
        j = pl.program_id(0)
        my_i = lax.axis_index("i")

        @pl.when(j == 0)
        def _comm():
            barrier_sem = pltpu.get_barrier_semaphore()
            for d in range(1, N_DEV):
                peer = lax.rem(my_i + d, N_DEV)
                pl.semaphore_signal(
                    barrier_sem, inc=1,
                    device_id=(peer,), device_id_type=pl.DeviceIdType.MESH,
                )
            xq_ref[...] = x_ref[...].astype(jnp.float8_e5m2)
            pl.semaphore_wait(barrier_sem, N_DEV - 1)

            for d in range(1, N_DEV):
                dst = lax.rem(my_i + d, N_DEV)
                rdma = pltpu.make_async_remote_copy(
                    src_ref=xq_ref.at[pl.ds(dst * MBLK, MBLK), :],
                    dst_ref=xg_ref.at[:, pl.ds(my_i * MBLK, MBLK)],
                    send_sem=send_sems.at[d],
                    recv_sem=recv_sems.at[d],
                    device_id=(dst,),
                    device_id_type=pl.DeviceIdType.MESH,
                )
                rdma.start()
            xg_ref[:, pl.ds(my_i * MBLK, MBLK)] = xq_ref[pl.ds(my_i * MBLK, MBLK), :]
            for d in range(1, N_DEV):
                src = lax.rem(my_i + (N_DEV - d), N_DEV)
                recv = pltpu.make_async_remote_copy(
                    src_ref=xq_ref.at[pl.ds(src * MBLK, MBLK), :],
                    dst_ref=xg_ref.at[:, pl.ds(src * MBLK, MBLK)],
                    send_sem=send_sems.at[d],
                    recv_sem=recv_sems.at[d],
                    device_id=(src,),
                    device_id_type=pl.DeviceIdType.MESH,
                )
                recv.wait_recv()
            xb_ref[...] = xg_ref[...].astype(jnp.bfloat16)

        wb = w_ref[...].astype(jnp.bfloat16)
        acc = jnp.dot(xb_ref[...], wb, preferred_element_type=jnp.float32)
        s = sx_ref[0] * sw_ref[0]
        y = acc * s
        out_ref[...] = y * jax.nn.sigmoid(jnp.clip(y, -60.0, 60.0))

        @pl.when(j == n_steps - 1)
        def _drain():
            for d in range(1, N_DEV):
                dst = lax.rem(my_i + d, N_DEV)
                send = pltpu.make_async_remote_copy(
                    src_ref=xq_ref.at[pl.ds(dst * MBLK, MBLK), :],
                    dst_ref=xg_ref.at[:, pl.ds(my_i * MBLK, MBLK)],
                    send_sem=send_sems.at[d],
                    recv_sem=recv_sems.at[d],
                    device_id=(dst,),
                    device_id_type=pl.DeviceIdType.MESH,
                )
                send.wait_send()

    return pl.pallas_call(
        body,
        grid=(n_steps,),
        out_shape=jax.ShapeDtypeStruct((MBLK, n), jnp.float32),
        in_specs=[
            pl.BlockSpec((m, k_shard), lambda j: (0, 0)),
            pl.BlockSpec((k_full, NBLK), lambda j: (0, j)),
            pl.BlockSpec(memory_space=pltpu.SMEM),
            pl.BlockSpec(memory_space=pltpu.SMEM),
        ],
        out_specs=pl.BlockSpec((MBLK, NBLK), lambda j: (0, j)),
        scratch_shapes=[
            pltpu.VMEM((m, k_shard), jnp.float8_e5m2),
            pltpu.VMEM((MBLK, k_full), jnp.float8_e5m2),
            pltpu.VMEM((MBLK, k_full), jnp.bfloat16),
            pltpu.SemaphoreType.DMA((N_DEV,)),
            pltpu.SemaphoreType.DMA((N_DEV,)),
        ],
        compiler_params=pltpu.CompilerParams(
            dimension_semantics=("arbitrary",),
            vmem_limit_bytes=60 * 1024 * 1024,
            collective_id=0,
        ),
    )(x, w_mat, scale_x, scale_w)


# device time: 58904 ns/iter; 1.2859x vs baseline; 1.1806x over previous
import jax
import jax.numpy as jnp
from jax import lax
from jax.experimental import pallas as pl
from jax.experimental.pallas import tpu as pltpu

N_DEV = 16
MBLK = 256
NBLK = 512
RING = 5


def kernel(x, w_mat, scale_x, scale_w):
    m, k_shard = x.shape
    k_full, n = w_mat.shape
    n_steps = n // NBLK
    assert m == N_DEV * MBLK and k_shard == MBLK and n % NBLK == 0

    def w_fetch(w_hbm, wv_ref, w_sems, blk, slot):
        return pltpu.make_async_copy(
            w_hbm.at[:, pl.ds(blk * NBLK, NBLK)],
            wv_ref.at[slot],
            w_sems.at[slot],
        )

    def body(x_ref, w_hbm, sx_ref, sw_ref, out_ref,
             wv_ref, xq_ref, xg_ref, xb_ref, w_sems, send_sems, recv_sems):
        j = pl.program_id(0)
        my_i = lax.axis_index("i")

        @pl.when(j == 0)
        def _comm():
            for r in range(RING):
                w_fetch(w_hbm, wv_ref, w_sems, r, r).start()

            barrier_sem = pltpu.get_barrier_semaphore()
            for d in range(1, N_DEV):
                peer = lax.rem(my_i + d, N_DEV)
                pl.semaphore_signal(
                    barrier_sem, inc=1,
                    device_id=(peer,), device_id_type=pl.DeviceIdType.MESH,
                )
            xq_ref[...] = x_ref[...].astype(jnp.float8_e5m2)
            pl.semaphore_wait(barrier_sem, N_DEV - 1)

            for d in range(1, N_DEV):
                dst = lax.rem(my_i + d, N_DEV)
                rdma = pltpu.make_async_remote_copy(
                    src_ref=xq_ref.at[pl.ds(dst * MBLK, MBLK), :],
                    dst_ref=xg_ref.at[:, pl.ds(my_i * MBLK, MBLK)],
                    send_sem=send_sems.at[d],
                    recv_sem=recv_sems.at[d],
                    device_id=(dst,),
                    device_id_type=pl.DeviceIdType.MESH,
                )
                rdma.start()
            xg_ref[:, pl.ds(my_i * MBLK, MBLK)] = xq_ref[pl.ds(my_i * MBLK, MBLK), :]
            for d in range(1, N_DEV):
                src = lax.rem(my_i + (N_DEV - d), N_DEV)
                recv = pltpu.make_async_remote_copy(
                    src_ref=xq_ref.at[pl.ds(src * MBLK, MBLK), :],
                    dst_ref=xg_ref.at[:, pl.ds(src * MBLK, MBLK)],
                    send_sem=send_sems.at[d],
                    recv_sem=recv_sems.at[d],
                    device_id=(src,),
                    device_id_type=pl.DeviceIdType.MESH,
                )
                recv.wait_recv()
            xb_ref[...] = xg_ref[...].astype(jnp.bfloat16)

        slot = lax.rem(j, RING)
        w_fetch(w_hbm, wv_ref, w_sems, j, slot).wait()
        wb = wv_ref[slot].astype(jnp.bfloat16)

        @pl.when(j + RING < n_steps)
        def _refill():
            w_fetch(w_hbm, wv_ref, w_sems, j + RING, slot).start()

        acc = jnp.dot(xb_ref[...], wb, preferred_element_type=jnp.float32)
        s = sx_ref[0] * sw_ref[0]
        y = acc * s
        out_ref[...] = y * jax.nn.sigmoid(jnp.clip(y, -60.0, 60.0))

        @pl.when(j == n_steps - 1)
        def _drain():
            for d in range(1, N_DEV):
                dst = lax.rem(my_i + d, N_DEV)
                send = pltpu.make_async_remote_copy(
                    src_ref=xq_ref.at[pl.ds(dst * MBLK, MBLK), :],
                    dst_ref=xg_ref.at[:, pl.ds(my_i * MBLK, MBLK)],
                    send_sem=send_sems.at[d],
                    recv_sem=recv_sems.at[d],
                    device_id=(dst,),
                    device_id_type=pl.DeviceIdType.MESH,
                )
                send.wait_send()

    return pl.pallas_call(
        body,
        grid=(n_steps,),
        out_shape=jax.ShapeDtypeStruct((MBLK, n), jnp.float32),
        in_specs=[
            pl.BlockSpec((m, k_shard), lambda j: (0, 0)),
            pl.BlockSpec(memory_space=pl.ANY),
            pl.BlockSpec(memory_space=pltpu.SMEM),
            pl.BlockSpec(memory_space=pltpu.SMEM),
        ],
        out_specs=pl.BlockSpec((MBLK, NBLK), lambda j: (0, j)),
        scratch_shapes=[
            pltpu.VMEM((RING, k_full, NBLK), jnp.float32),
            pltpu.VMEM((m, k_shard), jnp.float8_e5m2),
            pltpu.VMEM((MBLK, k_full), jnp.float8_e5m2),
            pltpu.VMEM((MBLK, k_full), jnp.bfloat16),
            pltpu.SemaphoreType.DMA((RING,)),
            pltpu.SemaphoreType.DMA((N_DEV,)),
            pltpu.SemaphoreType.DMA((N_DEV,)),
        ],
        compiler_params=pltpu.CompilerParams(
            dimension_semantics=("arbitrary",),
            vmem_limit_bytes=60 * 1024 * 1024,
            collective_id=0,
        ),
    )(x, w_mat, scale_x, scale_w)


# device time: 56013 ns/iter; 1.3523x vs baseline; 1.0516x over previous
import jax
import jax.numpy as jnp
from jax import lax
from jax.experimental import pallas as pl
from jax.experimental.pallas import tpu as pltpu

N_DEV = 16
MBLK = 256
NBLK = 512
RING = 6


def kernel(x, w_mat, scale_x, scale_w):
    m, k_shard = x.shape
    k_full, n = w_mat.shape
    n_steps = n // NBLK
    assert m == N_DEV * MBLK and k_shard == MBLK and n % NBLK == 0

    def w_fetch(w_hbm, wv_ref, w_sems, blk, slot):
        return pltpu.make_async_copy(
            w_hbm.at[:, pl.ds(blk * NBLK, NBLK)],
            wv_ref.at[slot],
            w_sems.at[slot],
        )

    def body(x_ref, w_hbm, sx_ref, sw_ref, out_ref,
             wv_ref, xq_ref, xg_ref, w_sems, send_sems, recv_sems):
        j = pl.program_id(0)
        my_i = lax.axis_index("i")

        @pl.when(j == 0)
        def _comm():
            for r in range(RING):
                w_fetch(w_hbm, wv_ref, w_sems, r, r).start()

            barrier_sem = pltpu.get_barrier_semaphore()
            for d in range(1, N_DEV):
                peer = lax.rem(my_i + d, N_DEV)
                pl.semaphore_signal(
                    barrier_sem, inc=1,
                    device_id=(peer,), device_id_type=pl.DeviceIdType.MESH,
                )
            xq_ref[...] = x_ref[...].astype(jnp.float8_e5m2)
            pl.semaphore_wait(barrier_sem, N_DEV - 1)

            for d in range(1, N_DEV):
                dst = lax.rem(my_i + d, N_DEV)
                rdma = pltpu.make_async_remote_copy(
                    src_ref=xq_ref.at[pl.ds(dst * MBLK, MBLK), :],
                    dst_ref=xg_ref.at[:, pl.ds(my_i * MBLK, MBLK)],
                    send_sem=send_sems.at[d],
                    recv_sem=recv_sems.at[d],
                    device_id=(dst,),
                    device_id_type=pl.DeviceIdType.MESH,
                )
                rdma.start()
            xg_ref[:, pl.ds(my_i * MBLK, MBLK)] = xq_ref[pl.ds(my_i * MBLK, MBLK), :]
            for d in range(1, N_DEV):
                src = lax.rem(my_i + (N_DEV - d), N_DEV)
                recv = pltpu.make_async_remote_copy(
                    src_ref=xq_ref.at[pl.ds(src * MBLK, MBLK), :],
                    dst_ref=xg_ref.at[:, pl.ds(src * MBLK, MBLK)],
                    send_sem=send_sems.at[d],
                    recv_sem=recv_sems.at[d],
                    device_id=(src,),
                    device_id_type=pl.DeviceIdType.MESH,
                )
                recv.wait_recv()

        slot = lax.rem(j, RING)
        w_fetch(w_hbm, wv_ref, w_sems, j, slot).wait()
        wb = wv_ref[slot].astype(jnp.float8_e5m2)

        @pl.when(j + RING < n_steps)
        def _refill():
            w_fetch(w_hbm, wv_ref, w_sems, j + RING, slot).start()

        acc = jnp.dot(xg_ref[...], wb, preferred_element_type=jnp.float32)
        s = sx_ref[0] * sw_ref[0]
        y = acc * s
        out_ref[...] = y * jax.nn.sigmoid(jnp.clip(y, -60.0, 60.0))

        @pl.when(j == n_steps - 1)
        def _drain():
            for d in range(1, N_DEV):
                dst = lax.rem(my_i + d, N_DEV)
                send = pltpu.make_async_remote_copy(
                    src_ref=xq_ref.at[pl.ds(dst * MBLK, MBLK), :],
                    dst_ref=xg_ref.at[:, pl.ds(my_i * MBLK, MBLK)],
                    send_sem=send_sems.at[d],
                    recv_sem=recv_sems.at[d],
                    device_id=(dst,),
                    device_id_type=pl.DeviceIdType.MESH,
                )
                send.wait_send()

    return pl.pallas_call(
        body,
        grid=(n_steps,),
        out_shape=jax.ShapeDtypeStruct((MBLK, n), jnp.float32),
        in_specs=[
            pl.BlockSpec((m, k_shard), lambda j: (0, 0)),
            pl.BlockSpec(memory_space=pl.ANY),
            pl.BlockSpec(memory_space=pltpu.SMEM),
            pl.BlockSpec(memory_space=pltpu.SMEM),
        ],
        out_specs=pl.BlockSpec((MBLK, NBLK), lambda j: (0, j)),
        scratch_shapes=[
            pltpu.VMEM((RING, k_full, NBLK), jnp.float32),
            pltpu.VMEM((m, k_shard), jnp.float8_e5m2),
            pltpu.VMEM((MBLK, k_full), jnp.float8_e5m2),
            pltpu.SemaphoreType.DMA((RING,)),
            pltpu.SemaphoreType.DMA((N_DEV,)),
            pltpu.SemaphoreType.DMA((N_DEV,)),
        ],
        compiler_params=pltpu.CompilerParams(
            dimension_semantics=("arbitrary",),
            vmem_limit_bytes=60 * 1024 * 1024,
            collective_id=0,
        ),
    )(x, w_mat, scale_x, scale_w)
